# Initial kernel scaffold; baseline (speedup 1.0000x reference)
import jax
import jax.numpy as jnp
from jax.experimental import pallas as pl


def kernel(patch_cls, patch_box, tgt_bbox, tgt_ids):
    raise NotImplementedError("write your pallas kernel here")


# speedup vs baseline: 2.9735x; 2.9735x over previous
import functools
import math

import jax
import jax.numpy as jnp
from jax.experimental import pallas as pl
from jax.experimental.pallas import tpu as pltpu

_ALPHA = 0.25
_EPS = 1e-7

_ATAN_COEF = (
    9.9999999999e-01, -3.3333332995e-01, 1.9999980354e-01, -1.4285262493e-01,
    1.1105656193e-01, -9.0511372748e-02, 7.5022314563e-02, -6.0385486713e-02,
    4.3902872330e-02, -2.6271578546e-02, 1.1602326100e-02, -3.2614871461e-03,
    4.3016498259e-04,
)


def _atan_pos(x):
    inv = x > 1.0
    z = jnp.where(inv, 1.0 / jnp.maximum(x, 1.0), x)
    u = z * z
    p = jnp.full_like(z, _ATAN_COEF[-1])
    for c in _ATAN_COEF[-2::-1]:
        p = p * u + c
    a = z * p
    return jnp.where(inv, (math.pi / 2.0) - a, a)


def _body(cls_ref, box_ref, tgt_ref, ids_ref, c_ref, am_ref, minv, mini, *, qb, nq):
    q = pl.program_id(1)

    cls = cls_ref[0]
    pb = box_ref[0]
    tb = tgt_ref[...]
    ids = ids_ref[...]
    n_cls = cls.shape[1]
    n_t = tb.shape[1]

    pc = jax.nn.sigmoid(cls)
    neg = (1.0 - _ALPHA) * (pc * pc) * (-jnp.log(1.0 - pc + 1e-8))
    pos = _ALPHA * ((1.0 - pc) * (1.0 - pc)) * (-jnp.log(pc + 1e-8))
    diff = pos - neg
    classes = jax.lax.broadcasted_iota(jnp.int32, (n_cls, n_t), 0)
    onehot = (classes == ids).astype(jnp.float32)
    cost_class = jax.lax.dot(
        diff, onehot,
        precision=jax.lax.Precision.HIGHEST,
        preferred_element_type=jnp.float32,
    )

    cxq, cyq, wq, hq = (pb[:, i : i + 1] for i in range(4))
    cxt, cyt, wt, ht = (tb[i : i + 1, :] for i in range(4))

    cost_bbox = (
        jnp.abs(cxq - cxt) + jnp.abs(cyq - cyt) + jnp.abs(wq - wt) + jnp.abs(hq - ht)
    )

    x1q, y1q, x2q, y2q = cxq - wq * 0.5, cyq - hq * 0.5, cxq + wq * 0.5, cyq + hq * 0.5
    x1t, y1t, x2t, y2t = cxt - wt * 0.5, cyt - ht * 0.5, cxt + wt * 0.5, cyt + ht * 0.5

    iw = jnp.maximum(jnp.minimum(x2q, x2t) - jnp.maximum(x1q, x1t), 0.0)
    ih = jnp.maximum(jnp.minimum(y2q, y2t) - jnp.maximum(y1q, y1t), 0.0)
    inter = iw * ih
    union = wq * hq + wt * ht - inter
    iou = inter / (union + _EPS)

    ew = jnp.maximum(jnp.maximum(x2q, x2t) - jnp.minimum(x1q, x1t), 0.0)
    eh = jnp.maximum(jnp.maximum(y2q, y2t) - jnp.minimum(y1q, y1t), 0.0)
    c2 = ew * ew + eh * eh + _EPS
    rho2 = (cxq - cxt) ** 2 + (cyq - cyt) ** 2

    atq = _atan_pos(wq / (hq + _EPS))
    att = _atan_pos(wt / (ht + _EPS))
    v = (4.0 / (math.pi ** 2)) * (atq - att) ** 2
    alpha = v / (1.0 - iou + v + _EPS)
    ciou = iou - rho2 / c2 - alpha * v

    c_tile = cost_bbox + cost_class - ciou
    c_ref[0] = c_tile

    colmin = jnp.min(c_tile, axis=0, keepdims=True)
    rows = jax.lax.broadcasted_iota(jnp.int32, c_tile.shape, 0)
    local_arg = (
        jnp.min(jnp.where(c_tile == colmin, rows, jnp.int32(qb)), axis=0, keepdims=True)
        + q * qb
    )

    @pl.when(q == 0)
    def _():
        minv[...] = colmin
        mini[...] = local_arg

    @pl.when(q > 0)
    def _():
        better = colmin < minv[...]
        mini[...] = jnp.where(better, local_arg, mini[...])
        minv[...] = jnp.where(better, colmin, minv[...])

    @pl.when(q == nq - 1)
    def _():
        am_ref[0] = mini[...]


def kernel(patch_cls, patch_box, tgt_bbox, tgt_ids):
    bs, qn, n_cls = patch_cls.shape
    n_t = tgt_bbox.shape[0]
    t_per = n_t // bs

    tgt_t = tgt_bbox.T
    ids2d = tgt_ids.reshape(1, n_t).astype(jnp.int32)

    qb = 1000
    nq = qn // qb
    grid = (bs, nq)

    body = functools.partial(_body, qb=qb, nq=nq)
    c_mat, am = pl.pallas_call(
        body,
        grid=grid,
        in_specs=[
            pl.BlockSpec((1, qb, n_cls), lambda b, q: (b, q, 0)),
            pl.BlockSpec((1, qb, 4), lambda b, q: (b, q, 0)),
            pl.BlockSpec((4, n_t), lambda b, q: (0, 0)),
            pl.BlockSpec((1, n_t), lambda b, q: (0, 0)),
        ],
        out_specs=[
            pl.BlockSpec((1, qb, n_t), lambda b, q: (b, q, 0)),
            pl.BlockSpec((1, 1, n_t), lambda b, q: (b, 0, 0)),
        ],
        out_shape=[
            jax.ShapeDtypeStruct((bs, qn, n_t), jnp.float32),
            jax.ShapeDtypeStruct((bs, 1, n_t), jnp.int32),
        ],
        scratch_shapes=[
            pltpu.VMEM((1, n_t), jnp.float32),
            pltpu.VMEM((1, n_t), jnp.int32),
        ],
    )(patch_cls, patch_box, tgt_t, ids2d)

    row_ind = jnp.stack([am[i, 0, i * t_per : (i + 1) * t_per] for i in range(bs)])
    col_ind = jnp.tile(jnp.arange(t_per, dtype=row_ind.dtype)[None, :], (bs, 1))
    return (c_mat, row_ind, col_ind)


# speedup vs baseline: 3.1896x; 1.0727x over previous
import functools
import math

import jax
import jax.numpy as jnp
from jax.experimental import pallas as pl
from jax.experimental.pallas import tpu as pltpu

_ALPHA = 0.25
_EPS = 1e-7

_ATAN_COEF = (
    9.9999999999e-01, -3.3333332995e-01, 1.9999980354e-01, -1.4285262493e-01,
    1.1105656193e-01, -9.0511372748e-02, 7.5022314563e-02, -6.0385486713e-02,
    4.3902872330e-02, -2.6271578546e-02, 1.1602326100e-02, -3.2614871461e-03,
    4.3016498259e-04,
)


def _atan_pos(x):
    inv = x > 1.0
    z = jnp.where(inv, 1.0 / jnp.maximum(x, 1.0), x)
    u = z * z
    p = jnp.full_like(z, _ATAN_COEF[-1])
    for c in _ATAN_COEF[-2::-1]:
        p = p * u + c
    a = z * p
    return jnp.where(inv, (math.pi / 2.0) - a, a)


def _body(cls_ref, box_ref, tgt_ref, ids_ref, c_ref, am_ref, minv, mini, *, qb, nq):
    q = pl.program_id(1)

    cls = cls_ref[0]
    pb = box_ref[0]
    tb = tgt_ref[...]
    ids = ids_ref[...]
    n_cls = cls.shape[1]
    n_t = tb.shape[1]

    pc = jax.nn.sigmoid(cls)
    neg = (1.0 - _ALPHA) * (pc * pc) * (-jnp.log(1.0 - pc + 1e-8))
    pos = _ALPHA * ((1.0 - pc) * (1.0 - pc)) * (-jnp.log(pc + 1e-8))
    diff = pos - neg
    classes = jax.lax.broadcasted_iota(jnp.int32, (n_cls, n_t), 0)
    onehot = (classes == ids).astype(jnp.float32)
    cost_class = jax.lax.dot(
        diff, onehot,
        precision=jax.lax.Precision.HIGHEST,
        preferred_element_type=jnp.float32,
    )

    cxq, cyq, wq, hq = (pb[:, i : i + 1] for i in range(4))
    cxt, cyt, wt, ht = (tb[i : i + 1, :] for i in range(4))
    hwq, hhq = 0.5 * wq, 0.5 * hq
    hwt, hht = 0.5 * wt, 0.5 * ht
    areaq = wq * hq
    areat = wt * ht
    atq = (2.0 / math.pi) * _atan_pos(wq / (hq + _EPS))
    att = (2.0 / math.pi) * _atan_pos(wt / (ht + _EPS))

    dx = cxq - cxt
    dy = cyq - cyt
    adx = jnp.abs(dx)
    ady = jnp.abs(dy)
    dwh = hwq - hwt
    dhh = hhq - hht
    adwh = jnp.abs(dwh)
    adhh = jnp.abs(dhh)
    cost_bbox = adx + ady + 2.0 * adwh + 2.0 * adhh

    sx = hwq + hwt
    sy = hhq + hht
    mx = jnp.maximum(adx, adwh)
    my = jnp.maximum(ady, adhh)
    iw = jnp.maximum(sx - mx, 0.0)
    ih = jnp.maximum(sy - my, 0.0)
    ew = sx + mx
    eh = sy + my
    inter = iw * ih
    iou = inter / (areaq + areat - inter + _EPS)

    c2 = ew * ew + eh * eh + _EPS
    rho2 = dx * dx + dy * dy

    dat = atq - att
    v = dat * dat
    denom = 1.0 - iou + v + _EPS
    ciou = iou - rho2 / c2 - (v * v) / denom

    c_tile = cost_bbox + cost_class - ciou
    c_ref[0] = c_tile

    colmin = jnp.min(c_tile, axis=0, keepdims=True)
    rows = jax.lax.broadcasted_iota(jnp.int32, c_tile.shape, 0)
    local_arg = (
        jnp.min(jnp.where(c_tile == colmin, rows, jnp.int32(qb)), axis=0, keepdims=True)
        + q * qb
    )

    @pl.when(q == 0)
    def _():
        minv[...] = colmin
        mini[...] = local_arg

    @pl.when(q > 0)
    def _():
        better = colmin < minv[...]
        mini[...] = jnp.where(better, local_arg, mini[...])
        minv[...] = jnp.where(better, colmin, minv[...])

    @pl.when(q == nq - 1)
    def _():
        am_ref[0] = mini[...]


def kernel(patch_cls, patch_box, tgt_bbox, tgt_ids):
    bs, qn, n_cls = patch_cls.shape
    n_t = tgt_bbox.shape[0]
    t_per = n_t // bs

    tgt_t = tgt_bbox.T
    ids2d = tgt_ids.reshape(1, n_t).astype(jnp.int32)

    qb = 1000
    nq = qn // qb
    grid = (bs, nq)

    body = functools.partial(_body, qb=qb, nq=nq)
    c_mat, am = pl.pallas_call(
        body,
        grid=grid,
        in_specs=[
            pl.BlockSpec((1, qb, n_cls), lambda b, q: (b, q, 0)),
            pl.BlockSpec((1, qb, 4), lambda b, q: (b, q, 0)),
            pl.BlockSpec((4, n_t), lambda b, q: (0, 0)),
            pl.BlockSpec((1, n_t), lambda b, q: (0, 0)),
        ],
        out_specs=[
            pl.BlockSpec((1, qb, n_t), lambda b, q: (b, q, 0)),
            pl.BlockSpec((1, 1, n_t), lambda b, q: (b, 0, 0)),
        ],
        out_shape=[
            jax.ShapeDtypeStruct((bs, qn, n_t), jnp.float32),
            jax.ShapeDtypeStruct((bs, 1, n_t), jnp.int32),
        ],
        scratch_shapes=[
            pltpu.VMEM((1, n_t), jnp.float32),
            pltpu.VMEM((1, n_t), jnp.int32),
        ],
    )(patch_cls, patch_box, tgt_t, ids2d)

    row_ind = jnp.stack([am[i, 0, i * t_per : (i + 1) * t_per] for i in range(bs)])
    col_ind = jnp.tile(jnp.arange(t_per, dtype=row_ind.dtype)[None, :], (bs, 1))
    return (c_mat, row_ind, col_ind)


# speedup vs baseline: 7.3970x; 2.3191x over previous
import math

import jax
import jax.numpy as jnp
from jax.experimental import pallas as pl

_ALPHA = 0.25
_EPS = 1e-7

_ATAN_COEF = (
    9.9999999999e-01, -3.3333332995e-01, 1.9999980354e-01, -1.4285262493e-01,
    1.1105656193e-01, -9.0511372748e-02, 7.5022314563e-02, -6.0385486713e-02,
    4.3902872330e-02, -2.6271578546e-02, 1.1602326100e-02, -3.2614871461e-03,
    4.3016498259e-04,
)


def _atan_pos(x):
    inv = x > 1.0
    z = jnp.where(inv, 1.0 / jnp.maximum(x, 1.0), x)
    u = z * z
    p = jnp.full_like(z, _ATAN_COEF[-1])
    for c in _ATAN_COEF[-2::-1]:
        p = p * u + c
    a = z * p
    return jnp.where(inv, (math.pi / 2.0) - a, a)


def _body(cls_ref, box_ref, tgt_ref, ids_ref, c_ref, am_ref):
    pb = box_ref[0]
    tb = tgt_ref[...]
    ids = ids_ref[...]
    n_q = pb.shape[1]
    n_cls = cls_ref.shape[1]

    cls = cls_ref[0]
    pc = 1.0 / (1.0 + jnp.exp(-cls))
    omp = 1.0 - pc
    neg = (pc * pc) * ((-(1.0 - _ALPHA)) * jnp.log(omp + 1e-8))
    pos = (omp * omp) * ((-_ALPHA) * jnp.log(pc + 1e-8))
    diff = pos - neg

    classes = jax.lax.broadcasted_iota(jnp.int32, (ids.shape[0], n_cls), 1)
    onehot = (classes == ids).astype(jnp.float32)
    cost_class = jax.lax.dot(
        onehot, diff,
        precision=jax.lax.Precision.HIGHEST,
        preferred_element_type=jnp.float32,
    )

    cxq, cyq, wq, hq = (pb[i : i + 1, :] for i in range(4))
    cxt, cyt, wt, ht = (tb[:, i : i + 1] for i in range(4))
    hwq, hhq = 0.5 * wq, 0.5 * hq
    hwt, hht = 0.5 * wt, 0.5 * ht
    areaq = wq * hq
    areat_eps = wt * ht + _EPS
    atq = (2.0 / math.pi) * _atan_pos(wq / (hq + _EPS))
    att = (2.0 / math.pi) * _atan_pos(wt / (ht + _EPS))

    dx = cxt - cxq
    dy = cyt - cyq
    adx = jnp.abs(dx)
    ady = jnp.abs(dy)
    dwh = hwt - hwq
    dhh = hht - hhq
    adwh = jnp.abs(dwh)
    adhh = jnp.abs(dhh)
    cost_bbox = (adx + ady) + 2.0 * (adwh + adhh)

    sx = hwt + hwq
    sy = hht + hhq
    mx = jnp.maximum(adx, adwh)
    my = jnp.maximum(ady, adhh)
    iw = jnp.maximum(sx - mx, 0.0)
    ih = jnp.maximum(sy - my, 0.0)
    ew = sx + mx
    eh = sy + my
    inter = iw * ih
    iou = inter / ((areat_eps + areaq) - inter)

    c2 = ew * ew + eh * eh + _EPS
    rho2 = dx * dx + dy * dy

    dat = att - atq
    v = dat * dat
    denom = (1.0 + _EPS) - iou + v
    ciou = iou - rho2 / c2 - (v * v) / denom

    c_tile = cost_bbox + cost_class - ciou
    c_ref[0] = c_tile

    rowmin = jnp.min(c_tile, axis=1, keepdims=True)
    cols = jax.lax.broadcasted_iota(jnp.int32, c_tile.shape, 1)
    am_ref[0] = jnp.min(
        jnp.where(c_tile == rowmin, cols, jnp.int32(n_q)), axis=1, keepdims=True
    )


def kernel(patch_cls, patch_box, tgt_bbox, tgt_ids):
    bs, qn, n_cls = patch_cls.shape
    n_t = tgt_bbox.shape[0]
    t_per = n_t // bs

    cls_t = jnp.swapaxes(patch_cls, 1, 2)
    box_t = jnp.swapaxes(patch_box, 1, 2)
    ids2d = tgt_ids.reshape(n_t, 1).astype(jnp.int32)

    c_t, am = pl.pallas_call(
        _body,
        grid=(bs,),
        in_specs=[
            pl.BlockSpec((1, n_cls, qn), lambda b: (b, 0, 0)),
            pl.BlockSpec((1, 4, qn), lambda b: (b, 0, 0)),
            pl.BlockSpec((n_t, 4), lambda b: (0, 0)),
            pl.BlockSpec((n_t, 1), lambda b: (0, 0)),
        ],
        out_specs=[
            pl.BlockSpec((1, n_t, qn), lambda b: (b, 0, 0)),
            pl.BlockSpec((1, n_t, 1), lambda b: (b, 0, 0)),
        ],
        out_shape=[
            jax.ShapeDtypeStruct((bs, n_t, qn), jnp.float32),
            jax.ShapeDtypeStruct((bs, n_t, 1), jnp.int32),
        ],
    )(cls_t, box_t, tgt_bbox, ids2d)

    c_mat = jnp.swapaxes(c_t, 1, 2)
    row_ind = jnp.stack([am[i, i * t_per : (i + 1) * t_per, 0] for i in range(bs)])
    col_ind = jnp.tile(jnp.arange(t_per, dtype=row_ind.dtype)[None, :], (bs, 1))
    return (c_mat, row_ind, col_ind)
